# baseline (speedup 1.0000x reference)
import functools
import math

import jax
import jax.numpy as jnp
from jax import lax
from jax.experimental import pallas as pl
from jax.experimental.pallas import tpu as pltpu
from jax.experimental.pallas import tpu_sc as plsc

_EMBED = 128
_LANES = 16
_NC, _NS = 2, 16

_PER_W = 104
_CHUNKS = ((0, 24), (24, 24), (48, 24), (72, 32))

_TWO_PI = 6.283185307179586
_PI = 3.141592653589793
_LOG1E4_OVER_64 = math.log(10000.0) / 64.0


def _sincos_bias(b, seg_v):
    lane = lax.iota(jnp.int32, _LANES)
    kf = (lane >> 1).astype(jnp.float32)
    even = (lane & 1) == 0
    bf = b.astype(jnp.float32)
    bias = []
    for j in range(_EMBED // _LANES):
        k = kf + float(8 * j)
        x = bf * jnp.exp(k * (-_LOG1E4_OVER_64))
        y = lax.rem(x, jnp.float32(_TWO_PI))
        z = jnp.where(y > _PI, y - _TWO_PI, y)
        z2 = z * z
        sinv = z * (1.0 + z2 * (-1.0 / 6 + z2 * (1.0 / 120 + z2 * (
            -1.0 / 5040 + z2 * (1.0 / 362880 + z2 * (
                -1.0 / 39916800 + z2 * (1.0 / 6227020800)))))))
        cosv = 1.0 + z2 * (-0.5 + z2 * (1.0 / 24 + z2 * (
            -1.0 / 720 + z2 * (1.0 / 40320 + z2 * (
                -1.0 / 3628800 + z2 * (1.0 / 479001600 + z2 * (
                    -1.0 / 87178291200)))))))
        bias.append(jnp.where(even, sinv, cosv) + seg_v[pl.ds(j * _LANES, _LANES)])
    return bias


@functools.lru_cache(maxsize=None)
def _build_sc_kernel(b_dim: int, l_dim: int):
    mesh = plsc.VectorSubcoreMesh(core_axis_name="c", subcore_axis_name="s")

    @functools.partial(
        pl.kernel,
        out_type=jax.ShapeDtypeStruct((b_dim, l_dim, _EMBED), jnp.float32),
        mesh=mesh,
        scratch_types=[
            pltpu.VMEM((8, 1, l_dim), jnp.int32),
            pltpu.VMEM((_PER_W,), jnp.int32),
            pltpu.VMEM((_EMBED,), jnp.float32),
            pltpu.VMEM((_PER_W, _EMBED), jnp.float32),
        ]
        + [pltpu.SemaphoreType.DMA] * (2 * len(_CHUNKS) + 1),
    )
    def sc_kernel(idx_hbm, seg_hbm, table_hbm, out_hbm,
                  idx_v, idx_row, seg_v, rows_v, *sems):
        nch = len(_CHUNKS)
        gsems, wsems, bsem = sems[:nch], sems[nch:2 * nch], sems[2 * nch]
        wid = lax.axis_index("s") * _NC + lax.axis_index("c")
        b = wid // 2
        r = b % 8
        l0 = (wid % 2) * (l_dim - _PER_W)
        seg_cp = pltpu.async_copy(seg_hbm.at[0], seg_v, bsem)
        pltpu.sync_copy(idx_hbm.at[pl.ds((b // 8) * 8, 8)], idx_v.at[:, 0, :])
        for o in (0, 16, 32, 48, 64, 80, _PER_W - _LANES):
            idx_row[pl.ds(o, _LANES)] = idx_v[r, 0, pl.ds(l0 + o, _LANES)]
        gathers = [
            pltpu.async_copy(
                table_hbm.at[idx_row.at[pl.ds(off, cnt)]],
                rows_v.at[pl.ds(off, cnt)],
                gsems[k],
            )
            for k, (off, cnt) in enumerate(_CHUNKS)
        ]
        seg_cp.wait()
        bias = _sincos_bias(b, seg_v)

        def add_row(i, carry):
            for j in range(_EMBED // _LANES):
                sl = pl.ds(j * _LANES, _LANES)
                rows_v[i, sl] = rows_v[i, sl] + bias[j]
            return carry

        writes = []
        for k, (off, cnt) in enumerate(_CHUNKS):
            gathers[k].wait()
            lax.fori_loop(off, off + cnt, add_row, 0, unroll=4)
            writes.append(
                pltpu.async_copy(
                    rows_v.at[pl.ds(off, cnt)],
                    out_hbm.at[b, pl.ds(l0 + off, cnt)],
                    wsems[k],
                )
            )
        for w in writes:
            w.wait()

    return sc_kernel


def kernel(sequence, token_table, segment_weight):
    B, L = sequence.shape
    idx = sequence.astype(jnp.int32)
    return _build_sc_kernel(B, L)(idx, segment_weight, token_table)


# speedup vs baseline: 1.0052x; 1.0052x over previous
import functools
import math

import jax
import jax.numpy as jnp
from jax import lax
from jax.experimental import pallas as pl
from jax.experimental.pallas import tpu as pltpu
from jax.experimental.pallas import tpu_sc as plsc

_EMBED = 128
_LANES = 16
_NC, _NS = 2, 16

_PER_W = 104
_CHUNKS = ((0, 24), (24, 24), (48, 24), (72, 32))

_TWO_PI = 6.283185307179586
_PI = 3.141592653589793
_LOG1E4_OVER_64 = math.log(10000.0) / 64.0


def _store_pe_row(b, bias_v):
    lane = lax.iota(jnp.int32, _LANES)
    kf = (lane >> 1).astype(jnp.float32)
    even = (lane & 1) == 0
    bf = b.astype(jnp.float32)

    def chunk(j, carry):
        k = kf + j.astype(jnp.float32) * 8.0
        x = bf * jnp.exp(k * (-_LOG1E4_OVER_64))
        y = lax.rem(x, jnp.float32(_TWO_PI))
        z = jnp.where(y > _PI, y - _TWO_PI, y)
        z2 = z * z
        sinv = z * (1.0 + z2 * (-1.0 / 6 + z2 * (1.0 / 120 + z2 * (
            -1.0 / 5040 + z2 * (1.0 / 362880 + z2 * (
                -1.0 / 39916800 + z2 * (1.0 / 6227020800)))))))
        cosv = 1.0 + z2 * (-0.5 + z2 * (1.0 / 24 + z2 * (
            -1.0 / 720 + z2 * (1.0 / 40320 + z2 * (
                -1.0 / 3628800 + z2 * (1.0 / 479001600 + z2 * (
                    -1.0 / 87178291200)))))))
        bias_v[pl.ds(j * _LANES, _LANES)] = jnp.where(even, sinv, cosv)
        return carry

    lax.fori_loop(0, _EMBED // _LANES, chunk, 0, unroll=2)


@functools.lru_cache(maxsize=None)
def _build_sc_kernel(b_dim: int, l_dim: int):
    mesh = plsc.VectorSubcoreMesh(core_axis_name="c", subcore_axis_name="s")

    @functools.partial(
        pl.kernel,
        out_type=jax.ShapeDtypeStruct((b_dim, l_dim, _EMBED), jnp.float32),
        mesh=mesh,
        scratch_types=[
            pltpu.VMEM((8, 1, l_dim), jnp.int32),
            pltpu.VMEM((_PER_W,), jnp.int32),
            pltpu.VMEM((_EMBED,), jnp.float32),
            pltpu.VMEM((_EMBED,), jnp.float32),
            pltpu.VMEM((_PER_W, _EMBED), jnp.float32),
        ]
        + [pltpu.SemaphoreType.DMA] * (2 * len(_CHUNKS) + 2),
    )
    def sc_kernel(idx_hbm, seg_hbm, table_hbm, out_hbm,
                  idx_v, idx_row, seg_v, bias_v, rows_v, *sems):
        nch = len(_CHUNKS)
        gsems, wsems, bsems = sems[:nch], sems[nch:2 * nch], sems[2 * nch:]
        wid = lax.axis_index("s") * _NC + lax.axis_index("c")
        b = wid // 2
        r = b % 8
        l0 = (wid % 2) * (l_dim - _PER_W)
        seg_cp = pltpu.async_copy(seg_hbm.at[0], seg_v, bsems[0])
        idx_cp = pltpu.async_copy(
            idx_hbm.at[pl.ds((b // 8) * 8, 8)], idx_v.at[:, 0, :], bsems[1])
        _store_pe_row(b, bias_v)
        idx_cp.wait()
        for o in (0, 16, 32, 48, 64, 80, _PER_W - _LANES):
            idx_row[pl.ds(o, _LANES)] = idx_v[r, 0, pl.ds(l0 + o, _LANES)]
        gathers = [
            pltpu.async_copy(
                table_hbm.at[idx_row.at[pl.ds(off, cnt)]],
                rows_v.at[pl.ds(off, cnt)],
                gsems[k],
            )
            for k, (off, cnt) in enumerate(_CHUNKS)
        ]
        seg_cp.wait()
        bias = [
            bias_v[pl.ds(j * _LANES, _LANES)] + seg_v[pl.ds(j * _LANES, _LANES)]
            for j in range(_EMBED // _LANES)
        ]

        def add_row(i, carry):
            for j in range(_EMBED // _LANES):
                sl = pl.ds(j * _LANES, _LANES)
                rows_v[i, sl] = rows_v[i, sl] + bias[j]
            return carry

        writes = []
        for k, (off, cnt) in enumerate(_CHUNKS):
            gathers[k].wait()
            lax.fori_loop(off, off + cnt, add_row, 0, unroll=4)
            writes.append(
                pltpu.async_copy(
                    rows_v.at[pl.ds(off, cnt)],
                    out_hbm.at[b, pl.ds(l0 + off, cnt)],
                    wsems[k],
                )
            )
        for w in writes:
            w.wait()

    return sc_kernel


def kernel(sequence, token_table, segment_weight):
    B, L = sequence.shape
    idx = sequence.astype(jnp.int32)
    return _build_sc_kernel(B, L)(idx, segment_weight, token_table)


# speedup vs baseline: 1.0467x; 1.0413x over previous
import functools
import math

import jax
import jax.numpy as jnp
import numpy as np
from jax import lax
from jax.experimental import pallas as pl
from jax.experimental.pallas import tpu as pltpu
from jax.experimental.pallas import tpu_sc as plsc

_EMBED = 128
_MAX_LEN = 30
_LANES = 16
_NC, _NS = 2, 16
_NW = _NC * _NS


def _pe_table() -> np.ndarray:
    position = np.arange(_MAX_LEN, dtype=np.float32)[:, None]
    div_term = np.exp(
        np.arange(0, _EMBED, 2, dtype=np.float32) * -(math.log(10000.0) / _EMBED)
    )
    pe = np.zeros((_MAX_LEN, _EMBED), dtype=np.float32)
    pe[:, 0::2] = np.sin(position * div_term)
    pe[:, 1::2] = np.cos(position * div_term)
    return pe


_PE = _pe_table()

_PER_W = 104
_CHUNKS = ((0, 32), (32, 32), (64, 32), (96, 8))


@functools.lru_cache(maxsize=None)
def _build_sc_kernel(n: int, half_l: int):
    mesh = plsc.VectorSubcoreMesh(core_axis_name="c", subcore_axis_name="s")

    @functools.partial(
        pl.kernel,
        out_type=jax.ShapeDtypeStruct((n, _EMBED), jnp.float32),
        mesh=mesh,
        scratch_types=[
            pltpu.VMEM((_PER_W,), jnp.int32),
            pltpu.VMEM((_EMBED,), jnp.float32),
            pltpu.VMEM((_EMBED,), jnp.float32),
            pltpu.VMEM((_PER_W, _EMBED), jnp.float32),
        ]
        + [pltpu.SemaphoreType.DMA] * (2 * len(_CHUNKS) + 2),
    )
    def sc_kernel(idx_hbm, pe_hbm, seg_hbm, table_hbm, out_hbm,
                  idx_v, pe_v, seg_v, rows_v, *sems):
        nch = len(_CHUNKS)
        gsems, wsems, bsems = sems[:nch], sems[nch:2 * nch], sems[2 * nch:]
        wid = lax.axis_index("s") * _NC + lax.axis_index("c")
        b = wid // 2
        base = b * (2 * half_l) + (wid % 2) * (2 * half_l - _PER_W)
        pe_cp = pltpu.async_copy(pe_hbm.at[b], pe_v, bsems[0])
        seg_cp = pltpu.async_copy(seg_hbm.at[0], seg_v, bsems[1])
        pltpu.sync_copy(idx_hbm.at[pl.ds(base, _PER_W)], idx_v)
        gathers = [
            pltpu.async_copy(
                table_hbm.at[idx_v.at[pl.ds(off, cnt)]],
                rows_v.at[pl.ds(off, cnt)],
                gsems[k],
            )
            for k, (off, cnt) in enumerate(_CHUNKS)
        ]
        pe_cp.wait()
        seg_cp.wait()
        bias = [
            pe_v[pl.ds(j * _LANES, _LANES)] + seg_v[pl.ds(j * _LANES, _LANES)]
            for j in range(_EMBED // _LANES)
        ]

        def add_row(i, carry):
            for j in range(_EMBED // _LANES):
                sl = pl.ds(j * _LANES, _LANES)
                rows_v[i, sl] = rows_v[i, sl] + bias[j]
            return carry

        writes = []
        for k, (off, cnt) in enumerate(_CHUNKS):
            gathers[k].wait()
            lax.fori_loop(off, off + cnt, add_row, 0, unroll=4)
            writes.append(
                pltpu.async_copy(
                    rows_v.at[pl.ds(off, cnt)],
                    out_hbm.at[pl.ds(base + off, cnt)],
                    wsems[k],
                )
            )
        for w in writes:
            w.wait()

    return sc_kernel


def kernel(sequence, token_table, segment_weight):
    B, L = sequence.shape
    idx = sequence.astype(jnp.int32).reshape(-1)
    pe = jnp.asarray(_PE[:B])
    out = _build_sc_kernel(B * L, L // 2)(idx, pe, segment_weight, token_table)
    return out.reshape(B, L, _EMBED)


# speedup vs baseline: 1.0498x; 1.0029x over previous
import functools
import math

import jax
import jax.numpy as jnp
import numpy as np
from jax import lax
from jax.experimental import pallas as pl
from jax.experimental.pallas import tpu as pltpu
from jax.experimental.pallas import tpu_sc as plsc

_EMBED = 128
_MAX_LEN = 30
_LANES = 16
_NC, _NS = 2, 16
_NW = _NC * _NS


def _pe_table() -> np.ndarray:
    position = np.arange(_MAX_LEN, dtype=np.float32)[:, None]
    div_term = np.exp(
        np.arange(0, _EMBED, 2, dtype=np.float32) * -(math.log(10000.0) / _EMBED)
    )
    pe = np.zeros((_MAX_LEN, _EMBED), dtype=np.float32)
    pe[:, 0::2] = np.sin(position * div_term)
    pe[:, 1::2] = np.cos(position * div_term)
    return pe


_PE = _pe_table()

_PER_W = 104
_CHUNKS = ((0, 16), (16, 40), (56, 40), (96, 8))


@functools.lru_cache(maxsize=None)
def _build_sc_kernel(n: int, half_l: int):
    mesh = plsc.VectorSubcoreMesh(core_axis_name="c", subcore_axis_name="s")

    @functools.partial(
        pl.kernel,
        out_type=jax.ShapeDtypeStruct((n, _EMBED), jnp.float32),
        mesh=mesh,
        scratch_types=[
            pltpu.VMEM((_PER_W,), jnp.int32),
            pltpu.VMEM((_EMBED,), jnp.float32),
            pltpu.VMEM((_EMBED,), jnp.float32),
            pltpu.VMEM((_PER_W, _EMBED), jnp.float32),
        ]
        + [pltpu.SemaphoreType.DMA] * (2 * len(_CHUNKS) + 2),
    )
    def sc_kernel(idx_hbm, pe_hbm, seg_hbm, table_hbm, out_hbm,
                  idx_v, pe_v, seg_v, rows_v, *sems):
        nch = len(_CHUNKS)
        gsems, wsems, bsems = sems[:nch], sems[nch:2 * nch], sems[2 * nch:]
        wid = lax.axis_index("s") * _NC + lax.axis_index("c")
        b = wid // 2
        base = b * (2 * half_l) + (wid % 2) * (2 * half_l - _PER_W)
        pe_cp = pltpu.async_copy(pe_hbm.at[b], pe_v, bsems[0])
        seg_cp = pltpu.async_copy(seg_hbm.at[0], seg_v, bsems[1])
        pltpu.sync_copy(idx_hbm.at[pl.ds(base, _PER_W)], idx_v)
        gathers = [
            pltpu.async_copy(
                table_hbm.at[idx_v.at[pl.ds(off, cnt)]],
                rows_v.at[pl.ds(off, cnt)],
                gsems[k],
            )
            for k, (off, cnt) in enumerate(_CHUNKS)
        ]
        pe_cp.wait()
        seg_cp.wait()
        bias = [
            pe_v[pl.ds(j * _LANES, _LANES)] + seg_v[pl.ds(j * _LANES, _LANES)]
            for j in range(_EMBED // _LANES)
        ]

        def add_row(i, carry):
            for j in range(_EMBED // _LANES):
                sl = pl.ds(j * _LANES, _LANES)
                rows_v[i, sl] = rows_v[i, sl] + bias[j]
            return carry

        writes = []
        for k, (off, cnt) in enumerate(_CHUNKS):
            gathers[k].wait()
            lax.fori_loop(off, off + cnt, add_row, 0, unroll=4)
            writes.append(
                pltpu.async_copy(
                    rows_v.at[pl.ds(off, cnt)],
                    out_hbm.at[pl.ds(base + off, cnt)],
                    wsems[k],
                )
            )
        for w in writes:
            w.wait()

    return sc_kernel


def kernel(sequence, token_table, segment_weight):
    B, L = sequence.shape
    idx = sequence.astype(jnp.int32).reshape(-1)
    pe = jnp.asarray(_PE[:B])
    out = _build_sc_kernel(B * L, L // 2)(idx, pe, segment_weight, token_table)
    return out.reshape(B, L, _EMBED)
